# baseline (speedup 1.0000x reference)
import functools

import jax
import jax.numpy as jnp
from jax import lax
from jax._src.pallas import core as pl_core
from jax.experimental import pallas as pl
from jax.experimental.pallas import tpu as pltpu
from jax.experimental.pallas import tpu_sc as plsc

_B, _C, _H, _W = 16, 85, 128, 128
_NGT = 50
_NCLS = _C - 5
_HW = _H * _W
_LB = 0.05
_W_OBJ = 1.0 / _HW
_W_CLS = 0.5 / (_HW * _NCLS)
_EPS = 1e-07

_NC, _NS, _L = 2, 16, 16
_NCHUNK = 4
_NPAD = _NCHUNK * _L


_LOG2E = 1.4426950408889634
_LN2 = 0.6931471805599453


def _dense_body(p_ref, out_ref):
    x = p_ref[...]
    xo = x[:, 0:1]
    xc = x[:, 1:]
    so = jnp.sum(jnp.log2(1.0 + jnp.exp2(xo * _LOG2E)))
    sc = jnp.sum(jnp.log2(1.0 + jnp.exp2(xc * _LOG2E)))
    val = (_LN2 * _W_OBJ) * so + (_LN2 * _W_CLS) * sc
    out_ref[0] = jnp.full((8, _W), 0.0) + val


def _dense_call(preds):
    return pl.pallas_call(
        _dense_body,
        grid=(_B,),
        in_specs=[pl.BlockSpec(
            (pl_core.Element(1), pl_core.Element(_C - 4),
             pl_core.Element(_H), pl_core.Element(_W)),
            lambda b: (b, 4, 0, 0),
        )],
        out_specs=pl.BlockSpec((1, 8, _W), lambda b: (b, 0, 0)),
        out_shape=jax.ShapeDtypeStruct((_B, 8, _W), jnp.float32),
        compiler_params=pltpu.CompilerParams(
            dimension_semantics=("parallel",)),
    )(preds)


def _sc_body(preds_hbm, tgt_hbm, out_hbm,
             ti, tgt_v, i0, i1, i2, i3, i4, i5,
             r0, r1, r2, r3, r4, r5,
             res_v, sem):
    w = lax.axis_index("s") * _NC + lax.axis_index("c")

    @pl.when(w < _B)
    def _work():
        b = w
        iota = lax.iota(jnp.int32, _L)
        tbase = b * (5 * _NGT)
        for f in range(5):
            for i in range(_NCHUNK):
                n = iota + (_L * i)
                g = jnp.minimum(n, _NGT - 1)
                ti[pl.ds(f * _NPAD + _L * i, _L)] = tbase + g * 5 + f
        pltpu.async_copy(tgt_hbm.at[ti], tgt_v, sem).wait()
        base = b * (_C * _HW)
        idx_refs = [i0, i1, i2, i3, i4, i5]
        row_refs = [r0, r1, r2, r3, r4, r5]

        zero_i = jnp.zeros((_L,), jnp.int32)
        one_i = jnp.ones((_L,), jnp.int32)
        tv_l, cell_l, key_l = [], [], []
        for i in range(_NCHUNK):
            n = iota + (_L * i)
            vmask_i = jnp.where(n < _NGT, one_i, zero_i)
            sent_i = jnp.where(n < _NGT, zero_i, -1 - n)
            sl = pl.ds(_L * i, _L)

            def tg(f, i=i):
                return tgt_v[pl.ds(f * _NPAD + _L * i, _L)]

            clsf = tg(0)
            cx = tg(1)
            cy = tg(2)
            gw = tg(3)
            gh = tg(4)
            cls_i = clsf.astype(jnp.int32)
            gi = (cx * float(_W)).astype(jnp.int32)
            gj = (cy * float(_H)).astype(jnp.int32)
            cell = gj * _W + gi
            cellc = jnp.minimum(cell, _HW - 1)
            for j in range(5):
                idx_refs[j][sl] = base + j * _HW + cellc
            idx_refs[5][sl] = base + (5 + cls_i) * _HW + cellc
            cell_l.append(cell * vmask_i + sent_i)
            key_l.append((cell * 128 + cls_i) * vmask_i + sent_i)
            tv_l.append((cx, cy, gw, gh))

        copies = [pltpu.async_copy(preds_hbm.at[idx_refs[j]], row_refs[j], sem)
                  for j in range(6)]

        dup_o = [zero_i] * _NCHUNK
        dup_c = [zero_i] * _NCHUNK
        for m in range(_NGT):
            ic, lane = m // _L, m % _L
            cmv = jnp.full((_L,), cell_l[ic][lane], jnp.int32)
            kmv = jnp.full((_L,), key_l[ic][lane], jnp.int32)
            lat_i = jnp.where(iota > (m - _L * ic), one_i, zero_i)
            dup_o[ic] += jnp.where(cell_l[ic] == cmv, lat_i, zero_i)
            dup_c[ic] += jnp.where(key_l[ic] == kmv, lat_i, zero_i)
            for i in range(ic + 1, _NCHUNK):
                dup_o[i] += jnp.where(cell_l[i] == cmv, one_i, zero_i)
                dup_c[i] += jnp.where(key_l[i] == kmv, one_i, zero_i)

        zerof = jnp.zeros((_L,), jnp.float32)
        onef = jnp.ones((_L,), jnp.float32)
        wbox_l, wo_l, wc_l = [], [], []
        for i in range(_NCHUNK):
            n = iota + (_L * i)
            vmask_f = jnp.where(n < _NGT, onef, zerof)
            wbox_l.append(vmask_f)
            wo_l.append(jnp.where(dup_o[i] == 0, vmask_f, zerof))
            wc_l.append(jnp.where(dup_c[i] == 0, vmask_f, zerof))

        for cp in copies:
            cp.wait()

        box_acc = jnp.zeros((_L,), jnp.float32)
        co_acc = jnp.zeros((_L,), jnp.float32)
        cc_acc = jnp.zeros((_L,), jnp.float32)
        for i in range(_NCHUNK):
            sl = pl.ds(_L * i, _L)
            cx, cy, gw, gh = tv_l[i]

            px = r0[sl]
            py = r1[sl]
            pw = r2[sl]
            ph = r3[sl]
            pobj = r4[sl]
            pcls = r5[sl]

            px1 = px - pw * 0.5
            py1 = py - ph * 0.5
            px2 = px + pw * 0.5
            py2 = py + ph * 0.5
            gx1 = (cx - gw * 0.5) * float(_W)
            gy1 = (cy - gh * 0.5) * float(_H)
            gx2 = (cx + gw * 0.5) * float(_W)
            gy2 = (cy + gh * 0.5) * float(_H)
            ix1 = jnp.maximum(px1, gx1)
            iy1 = jnp.maximum(py1, gy1)
            ix2 = jnp.minimum(px2, gx2)
            iy2 = jnp.minimum(py2, gy2)
            inter = jnp.maximum(ix2 - ix1, 0.0) * jnp.maximum(iy2 - iy1, 0.0)
            area1 = (px2 - px1) * (py2 - py1)
            area2 = (gx2 - gx1) * (gy2 - gy1)
            iou = inter / (area1 + area2 - inter + _EPS)

            box_acc += wbox_l[i] * (1.0 - iou)
            co_acc += wo_l[i] * pobj
            cc_acc += wc_l[i] * pcls

        res_v[...] = _LB * box_acc - _W_OBJ * co_acc - _W_CLS * cc_acc
        pltpu.sync_copy(res_v, out_hbm.at[w])


def _sc_call(preds_flat, targets_t):
    mesh = plsc.VectorSubcoreMesh(core_axis_name="c", subcore_axis_name="s")
    f = functools.partial(
        pl.kernel,
        mesh=mesh,
        out_type=jax.ShapeDtypeStruct((_B, _L), jnp.float32),
        scratch_types=(
            [pltpu.VMEM((5 * _NPAD,), jnp.int32)]
            + [pltpu.VMEM((5 * _NPAD,), jnp.float32)]
            + [pltpu.VMEM((_NPAD,), jnp.int32)] * 6
            + [pltpu.VMEM((_NPAD,), jnp.float32)] * 6
            + [pltpu.VMEM((_L,), jnp.float32)]
            + [pltpu.SemaphoreType.DMA]
        ),
    )(_sc_body)
    return f(preds_flat, targets_t)


def kernel(preds, targets):
    preds = preds.astype(jnp.float32)
    targets = targets.astype(jnp.float32)
    preds_flat = preds.reshape(_B * _C * _HW)
    targets_flat = targets.reshape(_B * _NGT * 5)
    dense = _dense_call(preds)
    parts = _sc_call(preds_flat, targets_flat)
    return jnp.sum(dense[:, 0, 0]) + jnp.sum(parts)


# speedup vs baseline: 1.0359x; 1.0359x over previous
import functools

import jax
import jax.numpy as jnp
from jax import lax
from jax._src.pallas import core as pl_core
from jax.experimental import pallas as pl
from jax.experimental.pallas import tpu as pltpu
from jax.experimental.pallas import tpu_sc as plsc

_B, _C, _H, _W = 16, 85, 128, 128
_NGT = 50
_NCLS = _C - 5
_HW = _H * _W
_LB = 0.05
_W_OBJ = 1.0 / _HW
_W_CLS = 0.5 / (_HW * _NCLS)
_EPS = 1e-07

_NC, _NS, _L = 2, 16, 16
_NCHUNK = 4
_NPAD = _NCHUNK * _L


_LOG2E = 1.4426950408889634
_LN2 = 0.6931471805599453


_IPB = 4


def _dense_body(p_ref, out_ref):
    x = p_ref[...]
    xo = x[:, 4:5]
    xc = x[:, 5:]
    so = jnp.sum(jnp.log2(1.0 + jnp.exp2(xo * _LOG2E)))
    sc = jnp.sum(jnp.log2(1.0 + jnp.exp2(xc * _LOG2E)))
    val = (_LN2 * _W_OBJ) * so + (_LN2 * _W_CLS) * sc
    out_ref[0] = jnp.full((8, _W), 0.0) + val


def _dense_call(preds):
    nblk = _B // _IPB
    return pl.pallas_call(
        _dense_body,
        grid=(nblk,),
        in_specs=[pl.BlockSpec((_IPB, _C, _H, _W), lambda b: (b, 0, 0, 0))],
        out_specs=pl.BlockSpec((1, 8, _W), lambda b: (b, 0, 0)),
        out_shape=jax.ShapeDtypeStruct((nblk, 8, _W), jnp.float32),
        compiler_params=pltpu.CompilerParams(
            dimension_semantics=("parallel",)),
    )(preds)


def _sc_body(preds_hbm, tgt_hbm, out_hbm,
             ti, tgt_v, i0, i1, i2, i3, i4, i5,
             r0, r1, r2, r3, r4, r5,
             res_v, sem):
    w = lax.axis_index("s") * _NC + lax.axis_index("c")

    @pl.when(w < _B)
    def _work():
        b = w
        iota = lax.iota(jnp.int32, _L)
        tbase = b * (5 * _NGT)
        for f in range(5):
            for i in range(_NCHUNK):
                n = iota + (_L * i)
                g = jnp.minimum(n, _NGT - 1)
                ti[pl.ds(f * _NPAD + _L * i, _L)] = tbase + g * 5 + f
        pltpu.async_copy(tgt_hbm.at[ti], tgt_v, sem).wait()
        base = b * (_C * _HW)
        idx_refs = [i0, i1, i2, i3, i4, i5]
        row_refs = [r0, r1, r2, r3, r4, r5]

        zero_i = jnp.zeros((_L,), jnp.int32)
        one_i = jnp.ones((_L,), jnp.int32)
        tv_l, cell_l, key_l = [], [], []
        for i in range(_NCHUNK):
            n = iota + (_L * i)
            vmask_i = jnp.where(n < _NGT, one_i, zero_i)
            sent_i = jnp.where(n < _NGT, zero_i, -1 - n)
            sl = pl.ds(_L * i, _L)

            def tg(f, i=i):
                return tgt_v[pl.ds(f * _NPAD + _L * i, _L)]

            clsf = tg(0)
            cx = tg(1)
            cy = tg(2)
            gw = tg(3)
            gh = tg(4)
            cls_i = clsf.astype(jnp.int32)
            gi = (cx * float(_W)).astype(jnp.int32)
            gj = (cy * float(_H)).astype(jnp.int32)
            cell = gj * _W + gi
            cellc = jnp.minimum(cell, _HW - 1)
            for j in range(5):
                idx_refs[j][sl] = base + j * _HW + cellc
            idx_refs[5][sl] = base + (5 + cls_i) * _HW + cellc
            cell_l.append(cell * vmask_i + sent_i)
            key_l.append((cell * 128 + cls_i) * vmask_i + sent_i)
            tv_l.append((cx, cy, gw, gh))

        copies = [pltpu.async_copy(preds_hbm.at[idx_refs[j]], row_refs[j], sem)
                  for j in range(6)]

        dup_o = [zero_i] * _NCHUNK
        dup_c = [zero_i] * _NCHUNK
        for m in range(_NGT):
            ic, lane = m // _L, m % _L
            cmv = jnp.full((_L,), cell_l[ic][lane], jnp.int32)
            kmv = jnp.full((_L,), key_l[ic][lane], jnp.int32)
            lat_i = jnp.where(iota > (m - _L * ic), one_i, zero_i)
            dup_o[ic] += jnp.where(cell_l[ic] == cmv, lat_i, zero_i)
            dup_c[ic] += jnp.where(key_l[ic] == kmv, lat_i, zero_i)
            for i in range(ic + 1, _NCHUNK):
                dup_o[i] += jnp.where(cell_l[i] == cmv, one_i, zero_i)
                dup_c[i] += jnp.where(key_l[i] == kmv, one_i, zero_i)

        zerof = jnp.zeros((_L,), jnp.float32)
        onef = jnp.ones((_L,), jnp.float32)
        wbox_l, wo_l, wc_l = [], [], []
        for i in range(_NCHUNK):
            n = iota + (_L * i)
            vmask_f = jnp.where(n < _NGT, onef, zerof)
            wbox_l.append(vmask_f)
            wo_l.append(jnp.where(dup_o[i] == 0, vmask_f, zerof))
            wc_l.append(jnp.where(dup_c[i] == 0, vmask_f, zerof))

        for cp in copies:
            cp.wait()

        box_acc = jnp.zeros((_L,), jnp.float32)
        co_acc = jnp.zeros((_L,), jnp.float32)
        cc_acc = jnp.zeros((_L,), jnp.float32)
        for i in range(_NCHUNK):
            sl = pl.ds(_L * i, _L)
            cx, cy, gw, gh = tv_l[i]

            px = r0[sl]
            py = r1[sl]
            pw = r2[sl]
            ph = r3[sl]
            pobj = r4[sl]
            pcls = r5[sl]

            px1 = px - pw * 0.5
            py1 = py - ph * 0.5
            px2 = px + pw * 0.5
            py2 = py + ph * 0.5
            gx1 = (cx - gw * 0.5) * float(_W)
            gy1 = (cy - gh * 0.5) * float(_H)
            gx2 = (cx + gw * 0.5) * float(_W)
            gy2 = (cy + gh * 0.5) * float(_H)
            ix1 = jnp.maximum(px1, gx1)
            iy1 = jnp.maximum(py1, gy1)
            ix2 = jnp.minimum(px2, gx2)
            iy2 = jnp.minimum(py2, gy2)
            inter = jnp.maximum(ix2 - ix1, 0.0) * jnp.maximum(iy2 - iy1, 0.0)
            area1 = (px2 - px1) * (py2 - py1)
            area2 = (gx2 - gx1) * (gy2 - gy1)
            iou = inter / (area1 + area2 - inter + _EPS)

            box_acc += wbox_l[i] * (1.0 - iou)
            co_acc += wo_l[i] * pobj
            cc_acc += wc_l[i] * pcls

        res_v[...] = _LB * box_acc - _W_OBJ * co_acc - _W_CLS * cc_acc
        pltpu.sync_copy(res_v, out_hbm.at[w])


def _sc_call(preds_flat, targets_t):
    mesh = plsc.VectorSubcoreMesh(core_axis_name="c", subcore_axis_name="s")
    f = functools.partial(
        pl.kernel,
        mesh=mesh,
        out_type=jax.ShapeDtypeStruct((_B, _L), jnp.float32),
        scratch_types=(
            [pltpu.VMEM((5 * _NPAD,), jnp.int32)]
            + [pltpu.VMEM((5 * _NPAD,), jnp.float32)]
            + [pltpu.VMEM((_NPAD,), jnp.int32)] * 6
            + [pltpu.VMEM((_NPAD,), jnp.float32)] * 6
            + [pltpu.VMEM((_L,), jnp.float32)]
            + [pltpu.SemaphoreType.DMA]
        ),
    )(_sc_body)
    return f(preds_flat, targets_t)


def kernel(preds, targets):
    preds = preds.astype(jnp.float32)
    targets = targets.astype(jnp.float32)
    preds_flat = preds.reshape(_B * _C * _HW)
    targets_flat = targets.reshape(_B * _NGT * 5)
    dense = _dense_call(preds)
    parts = _sc_call(preds_flat, targets_flat)
    return jnp.sum(dense[:, 0, 0]) + jnp.sum(parts)


# speedup vs baseline: 1.0563x; 1.0197x over previous
import functools

import jax
import jax.numpy as jnp
from jax import lax
from jax._src.pallas import core as pl_core
from jax.experimental import pallas as pl
from jax.experimental.pallas import tpu as pltpu
from jax.experimental.pallas import tpu_sc as plsc

_B, _C, _H, _W = 16, 85, 128, 128
_NGT = 50
_NCLS = _C - 5
_HW = _H * _W
_LB = 0.05
_W_OBJ = 1.0 / _HW
_W_CLS = 0.5 / (_HW * _NCLS)
_EPS = 1e-07

_NC, _NS, _L = 2, 16, 16
_NCHUNK = 4
_NPAD = _NCHUNK * _L


_LOG2E = 1.4426950408889634
_LN2 = 0.6931471805599453


_IPB = 2


def _dense_body(p_ref, out_ref):
    x = p_ref[...]
    xo = x[:, 4:5]
    xc = x[:, 5:]
    so = jnp.sum(jnp.log2(1.0 + jnp.exp2(xo * _LOG2E)))
    sc = jnp.sum(jnp.log2(1.0 + jnp.exp2(xc * _LOG2E)))
    val = (_LN2 * _W_OBJ) * so + (_LN2 * _W_CLS) * sc
    out_ref[0] = jnp.full((8, _W), 0.0) + val


def _dense_call(preds):
    nblk = _B // _IPB
    return pl.pallas_call(
        _dense_body,
        grid=(nblk,),
        in_specs=[pl.BlockSpec((_IPB, _C, _H, _W), lambda b: (b, 0, 0, 0))],
        out_specs=pl.BlockSpec((1, 8, _W), lambda b: (b, 0, 0)),
        out_shape=jax.ShapeDtypeStruct((nblk, 8, _W), jnp.float32),
        compiler_params=pltpu.CompilerParams(
            dimension_semantics=("parallel",)),
    )(preds)


def _sc_body(preds_hbm, tgt_hbm, out_hbm,
             ti, tgt_v, i0, i1, i2, i3, i4, i5,
             r0, r1, r2, r3, r4, r5,
             res_v, sem):
    w = lax.axis_index("s") * _NC + lax.axis_index("c")

    @pl.when(w < _B)
    def _work():
        b = w
        iota = lax.iota(jnp.int32, _L)
        tbase = b * (5 * _NGT)
        for f in range(5):
            for i in range(_NCHUNK):
                n = iota + (_L * i)
                g = jnp.minimum(n, _NGT - 1)
                ti[pl.ds(f * _NPAD + _L * i, _L)] = tbase + g * 5 + f
        pltpu.async_copy(tgt_hbm.at[ti], tgt_v, sem).wait()
        base = b * (_C * _HW)
        idx_refs = [i0, i1, i2, i3, i4, i5]
        row_refs = [r0, r1, r2, r3, r4, r5]

        zero_i = jnp.zeros((_L,), jnp.int32)
        one_i = jnp.ones((_L,), jnp.int32)
        tv_l, cell_l, key_l = [], [], []
        for i in range(_NCHUNK):
            n = iota + (_L * i)
            vmask_i = jnp.where(n < _NGT, one_i, zero_i)
            sent_i = jnp.where(n < _NGT, zero_i, -1 - n)
            sl = pl.ds(_L * i, _L)

            def tg(f, i=i):
                return tgt_v[pl.ds(f * _NPAD + _L * i, _L)]

            clsf = tg(0)
            cx = tg(1)
            cy = tg(2)
            gw = tg(3)
            gh = tg(4)
            cls_i = clsf.astype(jnp.int32)
            gi = (cx * float(_W)).astype(jnp.int32)
            gj = (cy * float(_H)).astype(jnp.int32)
            cell = gj * _W + gi
            cellc = jnp.minimum(cell, _HW - 1)
            for j in range(5):
                idx_refs[j][sl] = base + j * _HW + cellc
            idx_refs[5][sl] = base + (5 + cls_i) * _HW + cellc
            cell_l.append(cell * vmask_i + sent_i)
            key_l.append((cell * 128 + cls_i) * vmask_i + sent_i)
            tv_l.append((cx, cy, gw, gh))

        copies = [pltpu.async_copy(preds_hbm.at[idx_refs[j]], row_refs[j], sem)
                  for j in range(6)]

        dup_o = [zero_i] * _NCHUNK
        dup_c = [zero_i] * _NCHUNK
        for m in range(_NGT):
            ic, lane = m // _L, m % _L
            cmv = jnp.full((_L,), cell_l[ic][lane], jnp.int32)
            kmv = jnp.full((_L,), key_l[ic][lane], jnp.int32)
            lat_i = jnp.where(iota > (m - _L * ic), one_i, zero_i)
            dup_o[ic] += jnp.where(cell_l[ic] == cmv, lat_i, zero_i)
            dup_c[ic] += jnp.where(key_l[ic] == kmv, lat_i, zero_i)
            for i in range(ic + 1, _NCHUNK):
                dup_o[i] += jnp.where(cell_l[i] == cmv, one_i, zero_i)
                dup_c[i] += jnp.where(key_l[i] == kmv, one_i, zero_i)

        zerof = jnp.zeros((_L,), jnp.float32)
        onef = jnp.ones((_L,), jnp.float32)
        wbox_l, wo_l, wc_l = [], [], []
        for i in range(_NCHUNK):
            n = iota + (_L * i)
            vmask_f = jnp.where(n < _NGT, onef, zerof)
            wbox_l.append(vmask_f)
            wo_l.append(jnp.where(dup_o[i] == 0, vmask_f, zerof))
            wc_l.append(jnp.where(dup_c[i] == 0, vmask_f, zerof))

        for cp in copies:
            cp.wait()

        box_acc = jnp.zeros((_L,), jnp.float32)
        co_acc = jnp.zeros((_L,), jnp.float32)
        cc_acc = jnp.zeros((_L,), jnp.float32)
        for i in range(_NCHUNK):
            sl = pl.ds(_L * i, _L)
            cx, cy, gw, gh = tv_l[i]

            px = r0[sl]
            py = r1[sl]
            pw = r2[sl]
            ph = r3[sl]
            pobj = r4[sl]
            pcls = r5[sl]

            px1 = px - pw * 0.5
            py1 = py - ph * 0.5
            px2 = px + pw * 0.5
            py2 = py + ph * 0.5
            gx1 = (cx - gw * 0.5) * float(_W)
            gy1 = (cy - gh * 0.5) * float(_H)
            gx2 = (cx + gw * 0.5) * float(_W)
            gy2 = (cy + gh * 0.5) * float(_H)
            ix1 = jnp.maximum(px1, gx1)
            iy1 = jnp.maximum(py1, gy1)
            ix2 = jnp.minimum(px2, gx2)
            iy2 = jnp.minimum(py2, gy2)
            inter = jnp.maximum(ix2 - ix1, 0.0) * jnp.maximum(iy2 - iy1, 0.0)
            area1 = (px2 - px1) * (py2 - py1)
            area2 = (gx2 - gx1) * (gy2 - gy1)
            iou = inter / (area1 + area2 - inter + _EPS)

            box_acc += wbox_l[i] * (1.0 - iou)
            co_acc += wo_l[i] * pobj
            cc_acc += wc_l[i] * pcls

        res_v[...] = _LB * box_acc - _W_OBJ * co_acc - _W_CLS * cc_acc
        pltpu.sync_copy(res_v, out_hbm.at[w])


def _sc_call(preds_flat, targets_t):
    mesh = plsc.VectorSubcoreMesh(core_axis_name="c", subcore_axis_name="s")
    f = functools.partial(
        pl.kernel,
        mesh=mesh,
        out_type=jax.ShapeDtypeStruct((_B, _L), jnp.float32),
        scratch_types=(
            [pltpu.VMEM((5 * _NPAD,), jnp.int32)]
            + [pltpu.VMEM((5 * _NPAD,), jnp.float32)]
            + [pltpu.VMEM((_NPAD,), jnp.int32)] * 6
            + [pltpu.VMEM((_NPAD,), jnp.float32)] * 6
            + [pltpu.VMEM((_L,), jnp.float32)]
            + [pltpu.SemaphoreType.DMA]
        ),
    )(_sc_body)
    return f(preds_flat, targets_t)


def kernel(preds, targets):
    preds = preds.astype(jnp.float32)
    targets = targets.astype(jnp.float32)
    preds_flat = preds.reshape(_B * _C * _HW)
    targets_flat = targets.reshape(_B * _NGT * 5)
    dense = _dense_call(preds)
    parts = _sc_call(preds_flat, targets_flat)
    return jnp.sum(jnp.concatenate([dense[:, 0, 0], parts.reshape(-1)]))
